# Initial kernel scaffold; baseline (speedup 1.0000x reference)
import jax
import jax.numpy as jnp
from jax.experimental import pallas as pl


def kernel(z):
    raise NotImplementedError("write your pallas kernel here")


# speedup vs baseline: 4.5804x; 4.5804x over previous
import functools

import jax
import jax.numpy as jnp
from jax import lax
from jax.experimental import pallas as pl
from jax.experimental.pallas import tpu as pltpu

_K = 64
_N = 32768
_ROWS = 128
_RBLK = 8
_SIGN = -2147483648
_MASK31 = 0x7FFFFFFF


def _topk_block(z_ref, o_ref, s_ref):
    z = z_ref[...]
    b = lax.bitcast_convert_type(z, jnp.int32)
    s_ref[...] = b ^ (lax.shift_right_arithmetic(b, 31) & _MASK31)

    def val_step(i, v_u):
        bit = lax.shift_left(jnp.int32(1), jnp.int32(31) - i)
        c_u = v_u | bit
        c_s = c_u ^ _SIGN
        cnt = jnp.sum((s_ref[...] >= c_s).astype(jnp.int32), axis=1,
                      keepdims=True)
        return jnp.where(cnt >= _K, c_u, v_u)

    v_u = lax.fori_loop(0, 32, val_step, jnp.zeros((_RBLK, 1), jnp.int32))
    v_s = v_u ^ _SIGN

    sv = s_ref[...]
    cnt_gt = jnp.sum((sv > v_s).astype(jnp.int32), axis=1, keepdims=True)
    m = _K - cnt_gt
    cnt_eq = jnp.sum((sv == v_s).astype(jnp.int32), axis=1, keepdims=True)
    any_tie = jnp.any(cnt_eq > m)

    def tie_branch(_):
        def idx_step(i, c):
            t = c | lax.shift_left(jnp.int32(1), jnp.int32(16) - i)
            sv2 = s_ref[...]
            idx = lax.broadcasted_iota(jnp.int32, (_RBLK, _N), 1)
            g = jnp.sum(((sv2 == v_s) & (idx < t)).astype(jnp.int32),
                        axis=1, keepdims=True)
            return jnp.where(g <= m, t, c)

        return lax.fori_loop(0, 17, idx_step,
                             jnp.zeros((_RBLK, 1), jnp.int32))

    c_idx = lax.cond(any_tie, tie_branch,
                     lambda _: jnp.full((_RBLK, 1), _N, jnp.int32), None)

    idx = lax.broadcasted_iota(jnp.int32, (_RBLK, _N), 1)
    keep = (sv > v_s) | ((sv == v_s) & (idx < c_idx))
    o_ref[...] = jnp.where(keep, z, 0.0)


@jax.jit
def kernel(z):
    grid = (_ROWS // _RBLK,)
    return pl.pallas_call(
        _topk_block,
        grid=grid,
        in_specs=[pl.BlockSpec((_RBLK, _N), lambda i: (i, 0))],
        out_specs=pl.BlockSpec((_RBLK, _N), lambda i: (i, 0)),
        out_shape=jax.ShapeDtypeStruct((_ROWS, _N), jnp.float32),
        scratch_shapes=[pltpu.VMEM((_RBLK, _N), jnp.int32)],
        compiler_params=pltpu.CompilerParams(
            dimension_semantics=("parallel",)),
    )(z)


# speedup vs baseline: 6.5294x; 1.4255x over previous
import functools

import jax
import jax.numpy as jnp
from jax import lax
from jax.experimental import pallas as pl
from jax.experimental.pallas import tpu as pltpu

_K = 64
_N = 32768
_ROWS = 128
_RBLK = 8
_SIGN = -2147483648
_MASK31 = 0x7FFFFFFF


def _topk_block(z_ref, o_ref, s_ref):
    b = lax.bitcast_convert_type(z_ref[...], jnp.int32)
    s_ref[...] = b ^ (lax.shift_right_arithmetic(b, 31) & _MASK31)

    def count_ge(c_s):
        return jnp.sum((s_ref[...] >= c_s).astype(jnp.int32), axis=1,
                       keepdims=True)

    zeros = jnp.zeros((_RBLK, 1), jnp.int32)

    def search_cond(state):
        i, _, done, _ = state
        return jnp.logical_and(i < 32, jnp.logical_not(jnp.all(done == 1)))

    def search_body(state):
        i, v_u, done, thr = state
        bit = lax.shift_left(jnp.int32(1), jnp.int32(31) - i)
        c_u = v_u | bit
        cnt = count_ge(c_u ^ _SIGN)
        pinned = (cnt == _K) & (done == 0)
        thr = jnp.where(pinned, c_u, thr)
        done = done | pinned.astype(jnp.int32)
        v_u = jnp.where(cnt >= _K, c_u, v_u)
        return i + 1, v_u, done, thr

    _, v_u, done, thr = lax.while_loop(
        search_cond, search_body, (jnp.int32(0), zeros, zeros, zeros))

    v_u = jnp.where(done == 1, thr, v_u)
    v_s = v_u ^ _SIGN

    def tie_branch(_):
        sv = s_ref[...]
        cnt_gt = jnp.sum((sv > v_s).astype(jnp.int32), axis=1,
                         keepdims=True)
        m = _K - cnt_gt

        def idx_step(i, c):
            t = c | lax.shift_left(jnp.int32(1), jnp.int32(16) - i)
            sv2 = s_ref[...]
            idx = lax.broadcasted_iota(jnp.int32, (_RBLK, _N), 1)
            g = jnp.sum(((sv2 == v_s) & (idx < t)).astype(jnp.int32),
                        axis=1, keepdims=True)
            return jnp.where(g <= m, t, c)

        return lax.fori_loop(0, 17, idx_step, zeros)

    c_idx = lax.cond(jnp.all(done == 1),
                     lambda _: jnp.full((_RBLK, 1), _N, jnp.int32),
                     tie_branch, None)

    sv = s_ref[...]
    idx = lax.broadcasted_iota(jnp.int32, (_RBLK, _N), 1)
    keep = (sv > v_s) | ((sv == v_s) & (idx < c_idx))
    o_ref[...] = jnp.where(keep, z_ref[...], 0.0)


@jax.jit
def kernel(z):
    grid = (_ROWS // _RBLK,)
    return pl.pallas_call(
        _topk_block,
        grid=grid,
        in_specs=[pl.BlockSpec((_RBLK, _N), lambda i: (i, 0))],
        out_specs=pl.BlockSpec((_RBLK, _N), lambda i: (i, 0)),
        out_shape=jax.ShapeDtypeStruct((_ROWS, _N), jnp.float32),
        scratch_shapes=[pltpu.VMEM((_RBLK, _N), jnp.int32)],
        compiler_params=pltpu.CompilerParams(
            dimension_semantics=("parallel",)),
    )(z)


# speedup vs baseline: 13.9492x; 2.1364x over previous
import functools

import jax
import jax.numpy as jnp
from jax import lax
from jax.experimental import pallas as pl
from jax.experimental.pallas import tpu as pltpu

_K = 64
_N = 32768
_ROWS = 128
_RBLK = 16
_MASK31 = 0x7FFFFFFF
_BIAS30 = 1 << 30


def _sortable(z):
    b = lax.bitcast_convert_type(z, jnp.int32)
    return b ^ (lax.shift_right_arithmetic(b, 31) & _MASK31)


def _topk_block(z_ref, o_ref, s_ref):
    s_ref[...] = lax.shift_right_arithmetic(_sortable(z_ref[...]), 1)

    def count_ge(c_s):
        sv = s_ref[...]
        acc = lax.shift_right_arithmetic(sv[:, 0:1024] - c_s, 31)
        for a in range(1, _N // 1024):
            acc = acc + lax.shift_right_arithmetic(
                sv[:, a * 1024:(a + 1) * 1024] - c_s, 31)
        return _N + jnp.sum(acc, axis=1, keepdims=True)

    zeros = jnp.zeros((_RBLK, 1), jnp.int32)

    def search_cond(state):
        i, _, done, _ = state
        return jnp.logical_and(i < 31, jnp.logical_not(jnp.all(done == 1)))

    def search_body(state):
        i, v_u, done, thr = state
        bit = lax.shift_left(jnp.int32(1), jnp.int32(30) - i)
        c_u = v_u | bit
        cnt = count_ge(c_u - _BIAS30)
        pinned = (cnt == _K) & (done == 0)
        thr = jnp.where(pinned, c_u, thr)
        done = done | pinned.astype(jnp.int32)
        v_u = jnp.where(cnt >= _K, c_u, v_u)
        return i + 1, v_u, done, thr

    _, v_u, done, thr = lax.while_loop(
        search_cond, search_body, (jnp.int32(0), zeros, zeros, zeros))

    all_pinned = jnp.all(done == 1)

    @pl.when(all_pinned)
    def _fast():
        o_ref[...] = jnp.where(s_ref[...] >= thr - _BIAS30,
                               z_ref[...], 0.0)

    @pl.when(jnp.logical_not(all_pinned))
    def _general():
        sf = _sortable(z_ref[...])
        v31_s = v_u - _BIAS30
        cnt_gt31 = jnp.sum((s_ref[...] > v31_s).astype(jnp.int32),
                           axis=1, keepdims=True)
        m31 = _K - cnt_gt31
        cnt_odd = jnp.sum((sf == 2 * v31_s + 1).astype(jnp.int32),
                          axis=1, keepdims=True)
        v_full = jnp.where(cnt_odd >= m31, 2 * v31_s + 1, 2 * v31_s)
        v_full = jnp.where(done == 1, 2 * (thr - _BIAS30), v_full)

        cnt_gt = jnp.sum((sf > v_full).astype(jnp.int32), axis=1,
                         keepdims=True)
        m = _K - cnt_gt
        cnt_eq = jnp.sum((sf == v_full).astype(jnp.int32), axis=1,
                         keepdims=True)

        def tie_branch(_):
            def idx_step(i, c):
                t = c | lax.shift_left(jnp.int32(1), jnp.int32(16) - i)
                sv = _sortable(z_ref[...])
                idx = lax.broadcasted_iota(jnp.int32, (_RBLK, _N), 1)
                g = jnp.sum(((sv == v_full) & (idx < t)).astype(jnp.int32),
                            axis=1, keepdims=True)
                return jnp.where(g <= m, t, c)

            return lax.fori_loop(0, 17, idx_step, zeros)

        c_idx = lax.cond(jnp.all(cnt_eq <= m),
                         lambda _: jnp.full((_RBLK, 1), _N, jnp.int32),
                         tie_branch, None)

        idx = lax.broadcasted_iota(jnp.int32, (_RBLK, _N), 1)
        keep = (sf > v_full) | ((sf == v_full) & (idx < c_idx))
        o_ref[...] = jnp.where(keep, z_ref[...], 0.0)


@jax.jit
def kernel(z):
    grid = (_ROWS // _RBLK,)
    return pl.pallas_call(
        _topk_block,
        grid=grid,
        in_specs=[pl.BlockSpec((_RBLK, _N), lambda i: (i, 0))],
        out_specs=pl.BlockSpec((_RBLK, _N), lambda i: (i, 0)),
        out_shape=jax.ShapeDtypeStruct((_ROWS, _N), jnp.float32),
        scratch_shapes=[pltpu.VMEM((_RBLK, _N), jnp.int32)],
        compiler_params=pltpu.CompilerParams(
            dimension_semantics=("parallel",)),
    )(z)
